# baseline (speedup 1.0000x reference)
import functools

import jax
import jax.numpy as jnp
from jax import lax
from jax.experimental import pallas as pl
from jax.experimental.pallas import tpu as pltpu
from jax.experimental.pallas import tpu_sc as plsc

GRID = 64
D_ROW = 1024
D_COL = 1024
D_MODEL = D_ROW + D_COL
SEQ = GRID * GRID

NC = 2
NS = 16
NW = NC * NS
HB = GRID // 2


@functools.partial(
    pl.kernel,
    mesh=plsc.VectorSubcoreMesh(core_axis_name="c", subcore_axis_name="s"),
    out_type=jax.ShapeDtypeStruct((SEQ, D_MODEL), jnp.float32),
    scratch_types=[
        pltpu.VMEM((1, D_ROW), jnp.float32),
        pltpu.VMEM((HB, D_ROW), jnp.float32),
        pltpu.VMEM((HB, D_COL), jnp.float32),
        pltpu.VMEM((HB, D_COL), jnp.float32),
        pltpu.VMEM_SHARED((GRID, D_COL), jnp.float32),
        pltpu.SemaphoreType.DMA,
        pltpu.SemaphoreType.DMA,
        pltpu.SemaphoreType.DMA,
    ],
)
def _pos_enc_sc(row_hbm, col_hbm, out_hbm, rowbuf, left, col_a, col_b,
                col_sh, sem_c, sem_lw, sem_rw):
    sid = lax.axis_index("s")
    wid = sid * NC + lax.axis_index("c")

    @pl.when(sid == 0)
    def _():
        pltpu.sync_copy(col_hbm, col_sh)

    def replicate(j, _):
        off = pl.multiple_of(j * 16, 16)
        v = rowbuf[0, pl.ds(off, 16)]
        for i in range(HB):
            left[i, pl.ds(off, 16)] = v
        return 0

    r0base = pl.multiple_of(2 * wid * GRID, GRID)
    pltpu.sync_copy(row_hbm.at[pl.ds(2 * wid, 1)], rowbuf)
    lax.fori_loop(0, D_ROW // 16, replicate, 0)
    wl0 = pltpu.async_copy(
        left, out_hbm.at[pl.ds(r0base, HB), pl.ds(0, D_ROW)], sem_lw)
    wl1 = pltpu.async_copy(
        left, out_hbm.at[pl.ds(r0base + HB, HB), pl.ds(0, D_ROW)], sem_lw)

    plsc.subcore_barrier()
    cp_a = pltpu.async_copy(col_sh.at[pl.ds(0, HB)], col_a, sem_c)
    cp_a.wait()

    right_writes = []
    for t in range(2):
        rbase = pl.multiple_of((2 * wid + t) * GRID, GRID)
        if t == 0:
            right_writes.append(pltpu.async_copy(
                col_a, out_hbm.at[pl.ds(rbase, HB), pl.ds(D_ROW, D_COL)],
                sem_rw))
        else:
            right_writes.append(pltpu.async_copy(
                col_sh.at[pl.ds(0, HB)],
                out_hbm.at[pl.ds(rbase, HB), pl.ds(D_ROW, D_COL)], sem_rw))
        right_writes.append(pltpu.async_copy(
            col_sh.at[pl.ds(HB, HB)],
            out_hbm.at[pl.ds(rbase + HB, HB), pl.ds(D_ROW, D_COL)],
            sem_rw))

    r1base = pl.multiple_of((2 * wid + 1) * GRID, GRID)
    pltpu.sync_copy(row_hbm.at[pl.ds(2 * wid + 1, 1)], rowbuf)
    wl0.wait()
    wl1.wait()
    lax.fori_loop(0, D_ROW // 16, replicate, 0)
    wl2 = pltpu.async_copy(
        left, out_hbm.at[pl.ds(r1base, HB), pl.ds(0, D_ROW)], sem_lw)
    wl3 = pltpu.async_copy(
        left, out_hbm.at[pl.ds(r1base + HB, HB), pl.ds(0, D_ROW)], sem_lw)

    wl2.wait()
    wl3.wait()
    for w in right_writes:
        w.wait()


def kernel(seq_len, row_embed, col_embed):
    del seq_len
    return _pos_enc_sc(row_embed, col_embed)


# speedup vs baseline: 1.0209x; 1.0209x over previous
import functools

import jax
import jax.numpy as jnp
from jax import lax
from jax.experimental import pallas as pl
from jax.experimental.pallas import tpu as pltpu
from jax.experimental.pallas import tpu_sc as plsc

GRID = 64
D_ROW = 1024
D_COL = 1024
D_MODEL = D_ROW + D_COL
SEQ = GRID * GRID

NC = 2
NS = 16
NW = NC * NS
HB = GRID // 2


@functools.partial(
    pl.kernel,
    mesh=plsc.VectorSubcoreMesh(core_axis_name="c", subcore_axis_name="s"),
    out_type=jax.ShapeDtypeStruct((SEQ, D_MODEL), jnp.float32),
    scratch_types=[
        pltpu.VMEM((1, D_ROW), jnp.float32),
        pltpu.VMEM((HB, D_ROW), jnp.float32),
        pltpu.VMEM((HB, D_ROW), jnp.float32),
        pltpu.VMEM((HB, D_COL), jnp.float32),
        pltpu.VMEM_SHARED((GRID, D_COL), jnp.float32),
        pltpu.SemaphoreType.DMA,
        pltpu.SemaphoreType.DMA,
        pltpu.SemaphoreType.DMA,
    ],
)
def _pos_enc_sc(row_hbm, col_hbm, out_hbm, rowbuf, left_a, left_b, col_a,
                col_sh, sem_c, sem_lw, sem_rw):
    sid = lax.axis_index("s")
    wid = sid * NC + lax.axis_index("c")
    lefts = (left_a, left_b)

    @pl.when(sid == 0)
    def _():
        pltpu.sync_copy(col_hbm, col_sh)

    def make_fill(t):
        def fill(j, _):
            off = pl.multiple_of(j * 16, 16)
            v = rowbuf[0, pl.ds(off, 16)]
            for i in range(HB):
                lefts[t][i, pl.ds(off, 16)] = v
            return 0
        return fill

    left_writes = []
    r0base = pl.multiple_of(2 * wid * GRID, GRID)
    pltpu.sync_copy(row_hbm.at[pl.ds(2 * wid, 1)], rowbuf)
    lax.fori_loop(0, D_ROW // 16, make_fill(0), 0)
    left_writes.append(pltpu.async_copy(
        left_a, out_hbm.at[pl.ds(r0base, HB), pl.ds(0, D_ROW)], sem_lw))
    left_writes.append(pltpu.async_copy(
        left_a, out_hbm.at[pl.ds(r0base + HB, HB), pl.ds(0, D_ROW)], sem_lw))

    plsc.subcore_barrier()
    pltpu.async_copy(col_sh.at[pl.ds(0, HB)], col_a, sem_c).wait()

    right_writes = []
    for t in range(2):
        rbase = pl.multiple_of((2 * wid + t) * GRID, GRID)
        right_writes.append(pltpu.async_copy(
            col_a, out_hbm.at[pl.ds(rbase, HB), pl.ds(D_ROW, D_COL)], sem_rw))
        right_writes.append(pltpu.async_copy(
            col_sh.at[pl.ds(HB, HB)],
            out_hbm.at[pl.ds(rbase + HB, HB), pl.ds(D_ROW, D_COL)],
            sem_rw))

    r1base = pl.multiple_of((2 * wid + 1) * GRID, GRID)
    pltpu.sync_copy(row_hbm.at[pl.ds(2 * wid + 1, 1)], rowbuf)
    lax.fori_loop(0, D_ROW // 16, make_fill(1), 0)
    left_writes.append(pltpu.async_copy(
        left_b, out_hbm.at[pl.ds(r1base, HB), pl.ds(0, D_ROW)], sem_lw))
    left_writes.append(pltpu.async_copy(
        left_b, out_hbm.at[pl.ds(r1base + HB, HB), pl.ds(0, D_ROW)], sem_lw))

    for w in left_writes:
        w.wait()
    for w in right_writes:
        w.wait()


def kernel(seq_len, row_embed, col_embed):
    del seq_len
    return _pos_enc_sc(row_embed, col_embed)


# speedup vs baseline: 1.0298x; 1.0087x over previous
import functools

import jax
import jax.numpy as jnp
from jax import lax
from jax.experimental import pallas as pl
from jax.experimental.pallas import tpu as pltpu
from jax.experimental.pallas import tpu_sc as plsc

GRID = 64
D_ROW = 1024
D_COL = 1024
D_MODEL = D_ROW + D_COL
SEQ = GRID * GRID

NC = 2
NS = 16
NW = NC * NS
HB = GRID // 2


@functools.partial(
    pl.kernel,
    mesh=plsc.VectorSubcoreMesh(core_axis_name="c", subcore_axis_name="s"),
    out_type=jax.ShapeDtypeStruct((SEQ, D_MODEL), jnp.float32),
    scratch_types=[
        pltpu.VMEM((2, D_ROW), jnp.float32),
        pltpu.VMEM((HB, D_ROW), jnp.float32),
        pltpu.VMEM((HB, D_ROW), jnp.float32),
        pltpu.VMEM((HB, D_COL), jnp.float32),
        pltpu.VMEM_SHARED((GRID, D_COL), jnp.float32),
        pltpu.SemaphoreType.DMA,
        pltpu.SemaphoreType.DMA,
        pltpu.SemaphoreType.DMA,
    ],
)
def _pos_enc_sc(row_hbm, col_hbm, out_hbm, rowbuf, left_a, left_b, col_a,
                col_sh, sem_c, sem_lw, sem_rw):
    sid = lax.axis_index("s")
    wid = sid * NC + lax.axis_index("c")
    lefts = (left_a, left_b)

    cp_r0 = pltpu.async_copy(
        row_hbm.at[pl.ds(2 * wid, 1)], rowbuf.at[pl.ds(0, 1)], sem_c)
    cp_r1 = pltpu.async_copy(
        row_hbm.at[pl.ds(2 * wid + 1, 1)], rowbuf.at[pl.ds(1, 1)], sem_c)

    @pl.when(sid == 0)
    def _():
        pltpu.sync_copy(col_hbm, col_sh)

    def make_fill(t):
        def fill(j, _):
            off = pl.multiple_of(j * 16, 16)
            v = rowbuf[t, pl.ds(off, 16)]
            for i in range(HB):
                lefts[t][i, pl.ds(off, 16)] = v
            return 0
        return fill

    left_writes = []
    r0base = pl.multiple_of(2 * wid * GRID, GRID)
    cp_r0.wait()
    lax.fori_loop(0, D_ROW // 16, make_fill(0), 0)
    left_writes.append(pltpu.async_copy(
        left_a, out_hbm.at[pl.ds(r0base, HB), pl.ds(0, D_ROW)], sem_lw))
    left_writes.append(pltpu.async_copy(
        left_a, out_hbm.at[pl.ds(r0base + HB, HB), pl.ds(0, D_ROW)], sem_lw))

    plsc.subcore_barrier()
    pltpu.async_copy(col_sh.at[pl.ds(0, HB)], col_a, sem_c).wait()

    right_writes = []
    for t in range(2):
        rbase = pl.multiple_of((2 * wid + t) * GRID, GRID)
        right_writes.append(pltpu.async_copy(
            col_a, out_hbm.at[pl.ds(rbase, HB), pl.ds(D_ROW, D_COL)], sem_rw))
        right_writes.append(pltpu.async_copy(
            col_sh.at[pl.ds(HB, HB)],
            out_hbm.at[pl.ds(rbase + HB, HB), pl.ds(D_ROW, D_COL)],
            sem_rw))

    r1base = pl.multiple_of((2 * wid + 1) * GRID, GRID)
    cp_r1.wait()
    lax.fori_loop(0, D_ROW // 16, make_fill(1), 0)
    left_writes.append(pltpu.async_copy(
        left_b, out_hbm.at[pl.ds(r1base, HB), pl.ds(0, D_ROW)], sem_lw))
    left_writes.append(pltpu.async_copy(
        left_b, out_hbm.at[pl.ds(r1base + HB, HB), pl.ds(0, D_ROW)], sem_lw))

    for w in left_writes:
        w.wait()
    for w in right_writes:
        w.wait()


def kernel(seq_len, row_embed, col_embed):
    del seq_len
    return _pos_enc_sc(row_embed, col_embed)


# speedup vs baseline: 1.0298x; 1.0000x over previous
import functools

import jax
import jax.numpy as jnp
from jax import lax
from jax.experimental import pallas as pl
from jax.experimental.pallas import tpu as pltpu
from jax.experimental.pallas import tpu_sc as plsc

GRID = 64
D_ROW = 1024
D_COL = 1024
D_MODEL = D_ROW + D_COL
SEQ = GRID * GRID

NC = 2
NS = 16
NW = NC * NS
HB = GRID // 2


@functools.partial(
    pl.kernel,
    mesh=plsc.VectorSubcoreMesh(core_axis_name="c", subcore_axis_name="s"),
    out_type=jax.ShapeDtypeStruct((SEQ, D_MODEL), jnp.float32),
    scratch_types=[
        pltpu.VMEM((2, D_ROW), jnp.float32),
        pltpu.VMEM((HB, D_ROW), jnp.float32),
        pltpu.VMEM((HB, D_ROW), jnp.float32),
        pltpu.VMEM((HB, D_COL), jnp.float32),
        pltpu.VMEM_SHARED((GRID, D_COL), jnp.float32),
        pltpu.SemaphoreType.DMA,
        pltpu.SemaphoreType.DMA,
        pltpu.SemaphoreType.DMA,
        pltpu.SemaphoreType.DMA,
        pltpu.SemaphoreType.DMA,
    ],
)
def _pos_enc_sc(row_hbm, col_hbm, out_hbm, rowbuf, left_a, left_b, col_a,
                col_sh, sem_c, sem_r0, sem_r1, sem_lw, sem_rw):
    sid = lax.axis_index("s")
    wid = sid * NC + lax.axis_index("c")
    lefts = (left_a, left_b)

    cp_r0 = pltpu.async_copy(
        row_hbm.at[pl.ds(2 * wid, 1)], rowbuf.at[pl.ds(0, 1)], sem_r0)
    cp_r1 = pltpu.async_copy(
        row_hbm.at[pl.ds(2 * wid + 1, 1)], rowbuf.at[pl.ds(1, 1)], sem_r1)

    @pl.when(sid == 0)
    def _():
        pltpu.sync_copy(col_hbm, col_sh)

    def make_fill(t):
        def fill(j, _):
            off = pl.multiple_of(j * 16, 16)
            v = rowbuf[t, pl.ds(off, 16)]
            for i in range(HB):
                lefts[t][i, pl.ds(off, 16)] = v
            return 0
        return fill

    left_writes = []
    r0base = pl.multiple_of(2 * wid * GRID, GRID)
    cp_r0.wait()
    lax.fori_loop(0, D_ROW // 16, make_fill(0), 0)
    left_writes.append(pltpu.async_copy(
        left_a, out_hbm.at[pl.ds(r0base, HB), pl.ds(0, D_ROW)], sem_lw))
    left_writes.append(pltpu.async_copy(
        left_a, out_hbm.at[pl.ds(r0base + HB, HB), pl.ds(0, D_ROW)], sem_lw))

    plsc.subcore_barrier()
    pltpu.async_copy(col_sh.at[pl.ds(0, HB)], col_a, sem_c).wait()

    right_writes = []
    for t in range(2):
        rbase = pl.multiple_of((2 * wid + t) * GRID, GRID)
        right_writes.append(pltpu.async_copy(
            col_a, out_hbm.at[pl.ds(rbase, HB), pl.ds(D_ROW, D_COL)], sem_rw))
        right_writes.append(pltpu.async_copy(
            col_sh.at[pl.ds(HB, HB)],
            out_hbm.at[pl.ds(rbase + HB, HB), pl.ds(D_ROW, D_COL)],
            sem_rw))

    r1base = pl.multiple_of((2 * wid + 1) * GRID, GRID)
    cp_r1.wait()
    lax.fori_loop(0, D_ROW // 16, make_fill(1), 0)
    left_writes.append(pltpu.async_copy(
        left_b, out_hbm.at[pl.ds(r1base, HB), pl.ds(0, D_ROW)], sem_lw))
    left_writes.append(pltpu.async_copy(
        left_b, out_hbm.at[pl.ds(r1base + HB, HB), pl.ds(0, D_ROW)], sem_lw))

    for w in left_writes:
        w.wait()
    for w in right_writes:
        w.wait()


def kernel(seq_len, row_embed, col_embed):
    del seq_len
    return _pos_enc_sc(row_embed, col_embed)
